# baseline (device time: 48542 ns/iter reference)
import os

import jax
import jax.numpy as jnp
from jax import lax
from jax.experimental import pallas as pl
from jax.experimental.pallas import tpu as pltpu

_VARIANT = os.environ.get("KERNEL_VARIANT", "full")

B, SQ, SKV, H, D = 8, 1, 512, 8, 64
NY = 4
BH = B * H
HD = H * D
BK = B * SKV
SCALE = D ** -0.5


def kernel(Q, K, V):
    def body(q_ref, k_ref, v_ref, o_ref, comm_ref, send_sems, recv_sems):
        my_x = lax.axis_index("x")
        my_y = lax.axis_index("y")
        my_z = lax.axis_index("z")

        if _VARIANT == "loadonly":
            o_ref[...] = (k_ref[0:BH, 0:D] + v_ref[0:BH, 0:D]).reshape(
                B, SQ, H, D)
            return

        if _VARIANT == "full":
            barrier = pltpu.get_barrier_semaphore()
            for o in (1, 2, 3):
                pl.semaphore_signal(
                    barrier, inc=1,
                    device_id=(my_x, (my_y + o) % NY, my_z),
                    device_id_type=pl.DeviceIdType.MESH,
                )
            pl.semaphore_wait(barrier, 3)

        qt = jnp.transpose(q_ref[...])
        qr = jnp.broadcast_to(qt[:, :, None], (HD, B, H)).reshape(HD, BH)
        row_h = lax.broadcasted_iota(jnp.int32, (HD, BH), 0) // D
        col_h = lax.broadcasted_iota(jnp.int32, (HD, BH), 1) % H
        qbig = jnp.where(row_h == col_h, qr * SCALE, 0.0)
        s_big = jnp.dot(k_ref[...], qbig,
                        preferred_element_type=jnp.float32)
        p_big = jnp.exp(s_big)

        if _VARIANT == "qk":
            o_ref[...] = p_big[0:BH, 0:D].reshape(B, SQ, H, D)
            return

        pt = jnp.transpose(p_big)
        prow_b = lax.broadcasted_iota(jnp.int32, (BH, BK), 0) // H
        pcol_b = lax.broadcasted_iota(jnp.int32, (BH, BK), 1) // SKV
        p_bd = jnp.where(prow_b == pcol_b, pt, 0.0)
        den = jnp.sum(p_bd, axis=1, keepdims=True)
        numbig = jnp.dot(p_bd, v_ref[...],
                         preferred_element_type=jnp.float32)
        num3 = numbig.reshape(BH, H, D)
        nrow_h = lax.broadcasted_iota(jnp.int32, (BH, H), 0) % H
        ncol_h = lax.broadcasted_iota(jnp.int32, (BH, H), 1)
        hsel = jnp.where(nrow_h == ncol_h, 1.0, 0.0)
        num = jnp.sum(num3 * hsel[:, :, None], axis=1)

        if _VARIANT == "compute":
            o_ref[...] = (num / den).reshape(B, SQ, H, D)
            return

        comm_ref[0, :, 0:D] = num
        comm_ref[0, :, D:2 * D] = jnp.broadcast_to(den, (BH, D))

        rdmas = []
        for o in (1, 2, 3):
            rdma = pltpu.make_async_remote_copy(
                src_ref=comm_ref.at[0],
                dst_ref=comm_ref.at[o],
                send_sem=send_sems.at[o - 1],
                recv_sem=recv_sems.at[o - 1],
                device_id=(my_x, (my_y + o) % NY, my_z),
                device_id_type=pl.DeviceIdType.MESH,
            )
            rdma.start()
            rdmas.append(rdma)
        for rdma in rdmas:
            rdma.wait()

        tot = (comm_ref[0] + comm_ref[1] + comm_ref[2] + comm_ref[3])
        out = tot[:, 0:D] / tot[:, D:2 * D]
        o_ref[...] = out.reshape(B, SQ, H, D)

    return pl.pallas_call(
        body,
        out_shape=jax.ShapeDtypeStruct((B, SQ, H, D), jnp.float32),
        in_specs=[
            pl.BlockSpec(memory_space=pltpu.VMEM),
            pl.BlockSpec(memory_space=pltpu.VMEM),
            pl.BlockSpec(memory_space=pltpu.VMEM),
        ],
        out_specs=pl.BlockSpec(memory_space=pltpu.VMEM),
        scratch_shapes=[
            pltpu.VMEM((NY, BH, 2 * D), jnp.float32),
            pltpu.SemaphoreType.DMA((3,)),
            pltpu.SemaphoreType.DMA((3,)),
        ],
        compiler_params=pltpu.CompilerParams(
            collective_id=0,
            vmem_limit_bytes=100 * 1024 * 1024,
        ),
    )(Q.reshape(B, HD), K.reshape(BK, HD), V.reshape(BK, HD))


# device time: 40548 ns/iter; 1.1971x vs baseline; 1.1971x over previous
import os

import jax
import jax.numpy as jnp
from jax import lax
from jax.experimental import pallas as pl
from jax.experimental.pallas import tpu as pltpu

_VARIANT = os.environ.get("KERNEL_VARIANT", "full")

B, SQ, SKV, H, D = 8, 1, 512, 8, 64
NY = 4
BH = B * H
HD = H * D
BK = B * SKV
SCALE = D ** -0.5


def kernel(Q, K, V):
    def body(q_ref, k_ref, v_ref, o_ref, comm_ref, send_sems, recv_sems):
        my_x = lax.axis_index("x")
        my_y = lax.axis_index("y")
        my_z = lax.axis_index("z")

        if _VARIANT == "loadonly":
            o_ref[...] = (k_ref[0:BH, 0:D] + v_ref[0:BH, 0:D]).reshape(
                B, SQ, H, D)
            return

        if _VARIANT == "full":
            barrier = pltpu.get_barrier_semaphore()
            for o in (1, 2, 3):
                pl.semaphore_signal(
                    barrier, inc=1,
                    device_id=(my_x, (my_y + o) % NY, my_z),
                    device_id_type=pl.DeviceIdType.MESH,
                )
            pl.semaphore_wait(barrier, 3)

        qt = jnp.transpose(q_ref[...])
        qr = jnp.broadcast_to(qt[:, :, None], (HD, B, H)).reshape(HD, BH)
        row_h = lax.broadcasted_iota(jnp.int32, (HD, BH), 0) // D
        col_h = lax.broadcasted_iota(jnp.int32, (HD, BH), 1) % H
        qbig = jnp.where(row_h == col_h, qr * SCALE, 0.0)
        s_big = jnp.dot(k_ref[...], qbig,
                        preferred_element_type=jnp.float32)
        p_big = jnp.exp(s_big)

        if _VARIANT == "qk":
            o_ref[...] = p_big[0:BH, 0:D].reshape(B, SQ, H, D)
            return

        pt = jnp.transpose(p_big)
        prow_b = lax.broadcasted_iota(jnp.int32, (BH, BK), 0) // H
        pcol_b = lax.broadcasted_iota(jnp.int32, (BH, BK), 1) // SKV
        p_bd = jnp.where(prow_b == pcol_b, pt, 0.0)
        den = jnp.sum(p_bd, axis=1, keepdims=True)
        numbig = jnp.dot(p_bd, v_ref[...],
                         preferred_element_type=jnp.float32)
        num3 = numbig.reshape(BH, H, D)
        nrow_h = lax.broadcasted_iota(jnp.int32, (BH, H), 0) % H
        ncol_h = lax.broadcasted_iota(jnp.int32, (BH, H), 1)
        hsel = jnp.where(nrow_h == ncol_h, 1.0, 0.0)
        num = jnp.sum(num3 * hsel[:, :, None], axis=1)

        if _VARIANT == "compute":
            o_ref[...] = (num / den).reshape(B, SQ, H, D)
            return

        comm_ref[0, :, 0:D] = num
        comm_ref[0, :, D:2 * D] = jnp.broadcast_to(den, (BH, D))

        rdmas = []
        for o in (1, 2, 3):
            rdma = pltpu.make_async_remote_copy(
                src_ref=comm_ref.at[0],
                dst_ref=comm_ref.at[o],
                send_sem=send_sems.at[o - 1],
                recv_sem=recv_sems.at[o - 1],
                device_id=(my_x, (my_y + o) % NY, my_z),
                device_id_type=pl.DeviceIdType.MESH,
            )
            rdma.start()
            rdmas.append(rdma)
        for rdma in rdmas:
            rdma.wait()

        tot = (comm_ref[0] + comm_ref[1] + comm_ref[2] + comm_ref[3])
        out = tot[:, 0:D] / tot[:, D:2 * D]
        o_ref[...] = out.reshape(B, SQ, H, D)

    return pl.pallas_call(
        body,
        out_shape=jax.ShapeDtypeStruct((B, SQ, H, D), jnp.float32),
        in_specs=[
            pl.BlockSpec(memory_space=pltpu.VMEM),
            pl.BlockSpec(memory_space=pltpu.VMEM),
            pl.BlockSpec(memory_space=pltpu.VMEM),
        ],
        out_specs=pl.BlockSpec(memory_space=pltpu.VMEM),
        scratch_shapes=[
            pltpu.VMEM((NY, BH, 2 * D), jnp.float32),
            pltpu.SemaphoreType.DMA((3,)),
            pltpu.SemaphoreType.DMA((3,)),
        ],
        compiler_params=pltpu.CompilerParams(
            collective_id=0 if _VARIANT == "full" else None,
            vmem_limit_bytes=100 * 1024 * 1024,
        ),
    )(Q.reshape(B, HD), K.reshape(BK, HD), V.reshape(BK, HD))
